# Initial kernel scaffold; baseline (speedup 1.0000x reference)
import jax
import jax.numpy as jnp
from jax.experimental import pallas as pl


def kernel(DPTD_name_1, DPTD_name_2, table, W, b):
    raise NotImplementedError("write your pallas kernel here")


# speedup vs baseline: 1.2485x; 1.2485x over previous
import functools

import jax
import jax.numpy as jnp
from jax import lax
from jax.experimental import pallas as pl
from jax.experimental.pallas import tpu as pltpu
from jax.experimental.pallas import tpu_sc as plsc

_VOCAB = 1000
_EMBED = 20
_OUT = 10
_BATCH = 16384
_PAD = 16
_NC, _NS = 2, 16
_NW = _NC * _NS
_TOTAL = 2 * _BATCH
_PER_W = _TOTAL // _NW
_CHUNK = 128
_NCHUNK = _PER_W // _CHUNK


def _build_table_body(t_ref, w_ref, b_ref, o_ref):
    t = t_ref[...]
    w = w_ref[...]
    acc = jnp.broadcast_to(b_ref[...], (_VOCAB, _PAD))
    for k in range(_EMBED):
        acc = acc + t[:, k:k + 1] * w[k:k + 1, :]
    o_ref[...] = acc


def _gather_body(p_hbm, idx_hbm, out_hbm, idx_v, rows_v, sem):
    wid = lax.axis_index("s") * _NC + lax.axis_index("c")
    pltpu.sync_copy(idx_hbm.at[wid], idx_v)
    copies = []
    for j in range(_NCHUNK):
        copies.append(pltpu.async_copy(
            p_hbm.at[idx_v.at[j]],
            rows_v.at[pl.ds(j * _CHUNK, _CHUNK)],
            sem))
    for c in copies:
        c.wait()
    pltpu.sync_copy(rows_v, out_hbm.at[pl.ds(wid * _PER_W, _PER_W)])


def _sc_gather(p, idx3):
    mesh = plsc.VectorSubcoreMesh(core_axis_name="c", subcore_axis_name="s")
    run = pl.kernel(
        _gather_body,
        out_type=jax.ShapeDtypeStruct((_TOTAL, _PAD), jnp.float32),
        mesh=mesh,
        compiler_params=pltpu.CompilerParams(use_tc_tiling_on_sc=False),
        scratch_types=[
            pltpu.VMEM((_NCHUNK, _CHUNK), jnp.int32),
            pltpu.VMEM((_PER_W, _PAD), jnp.float32),
            pltpu.SemaphoreType.DMA,
        ],
    )
    return run(p, idx3)


def _reduce_body(y_ref, o_ref):
    y1 = y_ref[0:_BATCH, :]
    y2 = y_ref[_BATCH:_TOTAL, :]
    num = jnp.sum(y1 * y2, axis=0)
    s1 = jnp.sum(y1 * y1, axis=0)
    s2 = jnp.sum(y2 * y2, axis=0)
    denom = jnp.maximum(jnp.sqrt(s1) * jnp.sqrt(s2), 1e-8)
    o_ref[...] = (num / denom)[None, :]


def kernel(DPTD_name_1, DPTD_name_2, table, W, b):
    wt = jnp.zeros((_EMBED, _PAD), jnp.float32).at[:, :_OUT].set(W.T)
    bp = jnp.zeros((1, _PAD), jnp.float32).at[:, :_OUT].set(b[None, :])
    p = pl.pallas_call(
        _build_table_body,
        out_shape=jax.ShapeDtypeStruct((_VOCAB, _PAD), jnp.float32),
    )(table, wt, bp)
    idx = jnp.concatenate([DPTD_name_1, DPTD_name_2]).astype(jnp.int32)
    idx3 = idx.reshape(_NW, _NCHUNK, _CHUNK)
    y = _sc_gather(p, idx3)
    loss16 = pl.pallas_call(
        _reduce_body,
        out_shape=jax.ShapeDtypeStruct((1, _PAD), jnp.float32),
    )(y)
    loss = loss16[:, :_OUT]
    x1 = y[:_BATCH, :_OUT][None]
    x2 = y[_BATCH:, :_OUT][None]
    return loss, x1, x2


# speedup vs baseline: 1.3647x; 1.0931x over previous
import functools

import jax
import jax.numpy as jnp
from jax import lax
from jax.experimental import pallas as pl
from jax.experimental.pallas import tpu as pltpu
from jax.experimental.pallas import tpu_sc as plsc

_VOCAB = 1000
_EMBED = 20
_OUT = 10
_BATCH = 16384
_PAD = 16
_NC, _NS = 2, 16
_NW = _NC * _NS
_PER_W = _BATCH // _NW
_CHUNK = 128
_NCHUNK = _PER_W // _CHUNK


def _build_table_body(t_ref, w_ref, b_ref, o_ref):
    t = t_ref[...]
    w = w_ref[...]
    acc = jnp.broadcast_to(b_ref[...], (_VOCAB, _PAD))
    for k in range(_EMBED):
        acc = acc + t[:, k:k + 1] * w[k:k + 1, :]
    o_ref[...] = acc


def _gather_body(p_hbm, i1_hbm, i2_hbm, y1_hbm, y2_hbm,
                 i1_v, i2_v, r1_v, r2_v, sem, osem):
    wid = lax.axis_index("s") * _NC + lax.axis_index("c")
    base = wid * _PER_W
    pltpu.sync_copy(i1_hbm.at[wid], i1_v)
    pltpu.sync_copy(i2_hbm.at[wid], i2_v)
    copies = []
    for j in range(_NCHUNK):
        copies.append(pltpu.async_copy(
            p_hbm.at[i1_v.at[j]],
            r1_v.at[pl.ds(j * _CHUNK, _CHUNK)], sem))
        copies.append(pltpu.async_copy(
            p_hbm.at[i2_v.at[j]],
            r2_v.at[pl.ds(j * _CHUNK, _CHUNK)], sem))
    for c in copies:
        c.wait()
    out1 = pltpu.async_copy(r1_v, y1_hbm.at[pl.ds(base, _PER_W)], osem)
    out2 = pltpu.async_copy(r2_v, y2_hbm.at[pl.ds(base, _PER_W)], osem)
    out1.wait()
    out2.wait()


def _sc_gather(p, idx1r, idx2r):
    mesh = plsc.VectorSubcoreMesh(core_axis_name="c", subcore_axis_name="s")
    run = pl.kernel(
        _gather_body,
        out_type=(
            jax.ShapeDtypeStruct((_BATCH, _PAD), jnp.float32),
            jax.ShapeDtypeStruct((_BATCH, _PAD), jnp.float32),
        ),
        mesh=mesh,
        compiler_params=pltpu.CompilerParams(use_tc_tiling_on_sc=False),
        scratch_types=[
            pltpu.VMEM((_NCHUNK, _CHUNK), jnp.int32),
            pltpu.VMEM((_NCHUNK, _CHUNK), jnp.int32),
            pltpu.VMEM((_PER_W, _PAD), jnp.float32),
            pltpu.VMEM((_PER_W, _PAD), jnp.float32),
            pltpu.SemaphoreType.DMA,
            pltpu.SemaphoreType.DMA,
        ],
    )
    return run(p, idx1r, idx2r)


def _finish_body(y1_ref, y2_ref, loss_ref, x1_ref, x2_ref):
    a = y1_ref[...]
    c = y2_ref[...]
    num = jnp.sum(a * c, axis=0)
    s1 = jnp.sum(a * a, axis=0)
    s2 = jnp.sum(c * c, axis=0)
    denom = jnp.maximum(jnp.sqrt(s1) * jnp.sqrt(s2), 1e-8)
    loss_ref[...] = (num / denom)[None, :_OUT]
    x1_ref[...] = a[:, :_OUT]
    x2_ref[...] = c[:, :_OUT]


def kernel(DPTD_name_1, DPTD_name_2, table, W, b):
    wt = jnp.zeros((_EMBED, _PAD), jnp.float32).at[:, :_OUT].set(W.T)
    bp = jnp.zeros((1, _PAD), jnp.float32).at[:, :_OUT].set(b[None, :])
    p = pl.pallas_call(
        _build_table_body,
        out_shape=jax.ShapeDtypeStruct((_VOCAB, _PAD), jnp.float32),
    )(table, wt, bp)
    idx1r = DPTD_name_1.astype(jnp.int32).reshape(_NW, _NCHUNK, _CHUNK)
    idx2r = DPTD_name_2.astype(jnp.int32).reshape(_NW, _NCHUNK, _CHUNK)
    y1, y2 = _sc_gather(p, idx1r, idx2r)
    loss, x1, x2 = pl.pallas_call(
        _finish_body,
        out_shape=(
            jax.ShapeDtypeStruct((1, _OUT), jnp.float32),
            jax.ShapeDtypeStruct((_BATCH, _OUT), jnp.float32),
            jax.ShapeDtypeStruct((_BATCH, _OUT), jnp.float32),
        ),
    )(y1, y2)
    return loss, x1[None], x2[None]
